# baseline (speedup 1.0000x reference)
import functools

import jax
import jax.numpy as jnp
from jax import lax
from jax.experimental import pallas as pl
from jax.experimental.pallas import tpu as pltpu
from jax.experimental.pallas import tpu_sc as plsc

VOCAB = 28996
DIM = 768
NUM_WORKERS = 32
T_LEN = 20
B_LEN = 4096
B_TOTAL = B_LEN * T_LEN
COLS = B_LEN // NUM_WORKERS
CHUNK = 64
N_CHUNKS = T_LEN * COLS // CHUNK
N_PAIRS = N_CHUNKS // 2

_mesh = plsc.VectorSubcoreMesh(core_axis_name="c", subcore_axis_name="s")


@functools.partial(
    pl.kernel,
    mesh=_mesh,
    out_type=jax.ShapeDtypeStruct((B_TOTAL, DIM), jnp.float32),
    scratch_types=[
        pltpu.VMEM((T_LEN, COLS), jnp.int32),
        pltpu.VMEM((CHUNK, DIM), jnp.float32),
        pltpu.VMEM((CHUNK, DIM), jnp.float32),
        pltpu.SemaphoreType.DMA,
        pltpu.SemaphoreType.DMA,
        pltpu.SemaphoreType.DMA,
        pltpu.SemaphoreType.DMA,
    ],
)
def _gather_kernel(ids_hbm, table_hbm, out_hbm, idx_v, rows0, rows1,
                   gsem0, gsem1, ssem0, ssem1):
    wid = lax.axis_index("s") * 2 + lax.axis_index("c")
    col0 = wid * COLS
    pltpu.sync_copy(ids_hbm.at[:, pl.ds(col0, COLS)], idx_v)

    rows = (rows0, rows1)
    gsem = (gsem0, gsem1)
    ssem = (ssem0, ssem1)

    def idx_slice(t, h):
        return idx_v.at[t, pl.ds(h * CHUNK, CHUNK)]

    def out_slice(t, h):
        return out_hbm.at[pl.ds(t * B_LEN + col0 + h * CHUNK, CHUNK)]

    def start_gather(t, h):
        pltpu.async_copy(table_hbm.at[idx_slice(t, h)], rows[h], gsem[h])

    def wait_gather(t, h):
        pltpu.make_async_copy(
            table_hbm.at[idx_slice(t, h)], rows[h], gsem[h]
        ).wait()

    def start_scatter(t, h):
        pltpu.async_copy(rows[h], out_slice(t, h), ssem[h])

    def wait_scatter(t, h):
        pltpu.make_async_copy(rows[h], out_slice(t, h), ssem[h]).wait()

    start_gather(0, 0)
    start_gather(0, 1)

    def body(t, carry):
        for h in range(2):
            wait_gather(t, h)
            start_scatter(t, h)

            @pl.when(t < N_PAIRS - 1)
            def _():
                wait_scatter(t, h)
                start_gather(t + 1, h)

        return carry

    lax.fori_loop(0, N_PAIRS, body, 0)
    wait_scatter(N_PAIRS - 1, 0)
    wait_scatter(N_PAIRS - 1, 1)


def kernel(input_ids, embeddings):
    b, t = input_ids.shape
    ids2 = input_ids.T.astype(jnp.int32)
    out = _gather_kernel(ids2, embeddings)
    return out.reshape(t, b, DIM).transpose(1, 0, 2)
